# Initial kernel scaffold; baseline (speedup 1.0000x reference)
import jax
import jax.numpy as jnp
from jax.experimental import pallas as pl


def kernel(x, embed_weight, pe):
    raise NotImplementedError("write your pallas kernel here")


# speedup vs baseline: 3.2177x; 3.2177x over previous
import functools

import jax
import jax.numpy as jnp
from jax import lax
from jax.experimental import pallas as pl
from jax.experimental.pallas import tpu as pltpu
from jax.experimental.pallas import tpu_sc as plsc

B, P, V, D = 16384, 10, 14, 32
ROWS = B * P
NC, NS, L = 2, 16, 16
NW = NC * NS
ROWS_PER_W = ROWS // NW
CHUNK = 1280
NCHUNK = ROWS_PER_W // CHUNK


def _build_table(embed_weight, pe):
    def body(e_ref, p_ref, o_ref):
        o_ref[...] = e_ref[...][:, None, :] + p_ref[...][None, :, :]

    return pl.pallas_call(
        body,
        out_shape=jax.ShapeDtypeStruct((V, P, D), jnp.float32),
    )(embed_weight, pe)


def _make_gather():
    mesh = plsc.VectorSubcoreMesh(core_axis_name="c", subcore_axis_name="s")

    @functools.partial(
        pl.kernel,
        mesh=mesh,
        out_type=jax.ShapeDtypeStruct((ROWS, D), jnp.float32),
        scratch_types=[
            pltpu.VMEM((CHUNK,), jnp.int32),
            pltpu.VMEM((CHUNK,), jnp.int32),
            pltpu.VMEM((CHUNK, D), jnp.float32),
            pltpu.SemaphoreType.DMA,
        ],
        compiler_params=pltpu.CompilerParams(use_tc_tiling_on_sc=False),
    )
    def gather(table_hbm, x_hbm, out_hbm, x_v, idx_v, rows_v, sem):
        wid = lax.axis_index("s") * NC + lax.axis_index("c")

        def chunk_body(ci, _):
            base = wid * ROWS_PER_W + ci * CHUNK
            pltpu.sync_copy(x_hbm.at[pl.ds(base, CHUNK)], x_v)

            def vec_body(j, _):
                xv = x_v[pl.ds(j * L, L)]
                pos = (base + j * L + lax.iota(jnp.int32, L)) % P
                idx_v[pl.ds(j * L, L)] = xv * P + pos
                return 0

            lax.fori_loop(0, CHUNK // L, vec_body, 0)
            pltpu.async_copy(table_hbm.at[idx_v], rows_v, sem).wait()
            pltpu.sync_copy(rows_v, out_hbm.at[pl.ds(base, CHUNK)])
            return 0

        lax.fori_loop(0, NCHUNK, chunk_body, 0)

    return gather


_gather = _make_gather()


def kernel(x, embed_weight, pe):
    table = _build_table(embed_weight, pe).reshape(V * P, D)
    idx = x.reshape(ROWS).astype(jnp.int32)
    out = _gather(table, idx)
    return out.reshape(B, P, D)
